# Initial kernel scaffold; baseline (speedup 1.0000x reference)
import jax
import jax.numpy as jnp
from jax.experimental import pallas as pl


def kernel(inp_img, W1, b1, W2, b2, W3, b3):
    raise NotImplementedError("write your pallas kernel here")


# speedup vs baseline: 2.7311x; 2.7311x over previous
import jax
import jax.numpy as jnp
from jax.experimental import pallas as pl
from jax.experimental.pallas import tpu as pltpu
from functools import partial

N_BINS = 32
EPS = 6.4 / 256
LOW = -3.2 - EPS / 2
HIGH = 3.2 - EPS / 2
WIDTH = HIGH - LOW


def _hist_mlp_kernel(p0_ref, p1_ref, p2_ref, w1_ref, b1_ref, w2_ref, b2_ref,
                     w3_ref, b3_ref, out_ref):
    p0 = p0_ref[...]
    p1 = p1_ref[...]
    p2 = p2_ref[...]
    B, P = p0.shape

    valid = ((p0 > 0) & (p1 > 0) & (p2 > 0)).astype(jnp.float32)
    iy = jnp.sqrt(p0 * p0 + p1 * p1 + p2 * p2)
    s0 = jnp.where(p0 > 0, p0, 1.0)
    s1 = jnp.where(p1 > 0, p1, 1.0)
    s2 = jnp.where(p2 > 0, p2, 1.0)

    base_w = iy * valid
    bins = jax.lax.broadcasted_iota(jnp.int32, (B, P, N_BINS), 2)

    hist_parts = []
    for (si, su, sv) in ((s0, s2, s1), (s1, s2, s0), (s2, s1, s0)):
        iu = jnp.log(si / su)
        iv = jnp.log(si / sv)
        bu = jnp.floor((iu - LOW) / WIDTH * N_BINS).astype(jnp.int32)
        bv = jnp.floor((iv - LOW) / WIDTH * N_BINS).astype(jnp.int32)
        bu = jnp.where(iu == HIGH, N_BINS - 1, bu)
        bv = jnp.where(iv == HIGH, N_BINS - 1, bv)
        in_u = ((iu >= LOW) & (iu <= HIGH) & (bu >= 0) & (bu < N_BINS))
        in_v = ((iv >= LOW) & (iv <= HIGH) & (bv >= 0) & (bv < N_BINS))
        w = base_w * in_u.astype(jnp.float32) * in_v.astype(jnp.float32)

        u_oh = jnp.where(bu[:, :, None] == bins, w[:, :, None], 0.0)
        v_oh = jnp.where(bv[:, :, None] == bins, 1.0, 0.0)
        h2d = jax.lax.dot_general(
            u_oh, v_oh,
            dimension_numbers=(((1,), (1,)), ((0,), (0,))),
            preferred_element_type=jnp.float32,
        )
        h = h2d.reshape(B, N_BINS * N_BINS)
        norm = jnp.sum(h, axis=1, keepdims=True)
        hist_parts.append(jnp.sqrt(h / norm))

    hist = jnp.concatenate(hist_parts, axis=1)

    h1 = jax.lax.dot_general(hist, w1_ref[...], (((1,), (1,)), ((), ())),
                             preferred_element_type=jnp.float32)
    h1 = jnp.maximum(h1 + b1_ref[...][None, :], 0.0)
    h2 = jax.lax.dot_general(h1, w2_ref[...], (((1,), (1,)), ((), ())),
                             preferred_element_type=jnp.float32)
    h2 = jnp.maximum(h2 + b2_ref[...][None, :], 0.0)
    h3 = jax.lax.dot_general(h2, w3_ref[...], (((1,), (1,)), ((), ())),
                             preferred_element_type=jnp.float32)
    out_ref[...] = jnp.maximum(h3 + b3_ref[...][None, :], 0.0)


@jax.jit
def kernel(inp_img, W1, b1, W2, b2, W3, b3):
    B = inp_img.shape[0]
    H = inp_img.shape[2]
    stride = H // N_BINS
    small = inp_img[:, :, ::stride, ::stride]
    pix = small.reshape(B, 3, N_BINS * N_BINS)

    out = pl.pallas_call(
        _hist_mlp_kernel,
        out_shape=jax.ShapeDtypeStruct((B, W3.shape[0]), jnp.float32),
        compiler_params=pltpu.CompilerParams(
            vmem_limit_bytes=100 * 1024 * 1024,
        ),
    )(pix[:, 0], pix[:, 1], pix[:, 2], W1, b1, W2, b2, W3, b3)
    return out[:, :, None, None]
